# Initial kernel scaffold; baseline (speedup 1.0000x reference)
import jax
import jax.numpy as jnp
from jax.experimental import pallas as pl


def kernel(indices, codebook):
    raise NotImplementedError("write your pallas kernel here")


# speedup vs baseline: 3.3673x; 3.3673x over previous
import functools

import jax
import jax.numpy as jnp
from jax import lax
from jax.experimental import pallas as pl
from jax.experimental.pallas import tpu as pltpu
from jax.experimental.pallas import tpu_sc as plsc

_INFO = plsc.get_sparse_core_info()
_NC = _INFO.num_cores
_NS = _INFO.num_subcores
_NW = _NC * _NS

_B = 128 * 1024
_D = 64
_V = 8192
_C = 128
_NCHUNK = _B // _C
_CPW = _NCHUNK // _NW


def _gather_body(codebook_hbm, idx_hbm, out_hbm, idx_v, rows_v, sem):
    wid = lax.axis_index("s") * _NC + lax.axis_index("c")
    row0 = wid * _CPW
    pltpu.sync_copy(idx_hbm.at[pl.ds(row0, _CPW)], idx_v)

    def step(j, carry):
        pltpu.async_copy(codebook_hbm.at[idx_v.at[j]], rows_v, sem).wait()
        pltpu.sync_copy(rows_v, out_hbm.at[pl.ds((row0 + j) * _C, _C)])
        return carry

    lax.fori_loop(0, _CPW, step, 0)


@functools.partial(jax.jit, static_argnames=())
def _gather(codebook, idx2d):
    k = pl.kernel(
        _gather_body,
        out_type=jax.ShapeDtypeStruct((_B, _D), jnp.float32),
        mesh=plsc.VectorSubcoreMesh(core_axis_name="c", subcore_axis_name="s"),
        scratch_types=[
            pltpu.VMEM((_CPW, _C), jnp.int32),
            pltpu.VMEM((_C, _D), jnp.float32),
            pltpu.SemaphoreType.DMA,
        ],
        compiler_params=pltpu.CompilerParams(use_tc_tiling_on_sc=False),
    )
    return k(codebook, idx2d)


def kernel(indices, codebook):
    idx2d = indices.reshape(_NCHUNK, _C)
    out = _gather(codebook, idx2d)
    return out.reshape(indices.shape + (codebook.shape[-1],))


# speedup vs baseline: 3.8210x; 1.1348x over previous
import functools

import jax
import jax.numpy as jnp
from jax import lax
from jax.experimental import pallas as pl
from jax.experimental.pallas import tpu as pltpu
from jax.experimental.pallas import tpu_sc as plsc

_INFO = plsc.get_sparse_core_info()
_NC = _INFO.num_cores
_NS = _INFO.num_subcores
_NW = _NC * _NS

_B = 128 * 1024
_D = 64
_V = 8192
_C = 128
_NCHUNK = _B // _C
_CPW = _NCHUNK // _NW
_K = 4
_ROWS = _K * _C
_G = _CPW // _K


def _gather_body(codebook_hbm, idx_hbm, out_hbm,
                 idx_v, rows_a, rows_b, gsem, wsem_a, wsem_b):
    wid = lax.axis_index("s") * _NC + lax.axis_index("c")
    row0 = wid * _CPW
    pltpu.sync_copy(idx_hbm.at[pl.ds(row0, _CPW)], idx_v)

    def half(p, g, rows_v, wsem):
        @pl.when(p > 0)
        def _():
            pltpu.make_async_copy(
                rows_v, out_hbm.at[pl.ds(0, _ROWS)], wsem).wait()
        descs = [
            pltpu.async_copy(
                codebook_hbm.at[idx_v.at[g * _K + k]],
                rows_v.at[pl.ds(k * _C, _C)],
                gsem,
            )
            for k in range(_K)
        ]
        for d in descs:
            d.wait()
        pltpu.async_copy(
            rows_v, out_hbm.at[pl.ds((row0 + g * _K) * _C, _ROWS)], wsem)

    def step(p, carry):
        half(p, 2 * p, rows_a, wsem_a)
        half(p, 2 * p + 1, rows_b, wsem_b)
        return carry

    lax.fori_loop(0, _G // 2, step, 0)
    pltpu.make_async_copy(rows_a, out_hbm.at[pl.ds(0, _ROWS)], wsem_a).wait()
    pltpu.make_async_copy(rows_b, out_hbm.at[pl.ds(0, _ROWS)], wsem_b).wait()


@functools.partial(jax.jit, static_argnames=())
def _gather(codebook, idx2d):
    k = pl.kernel(
        _gather_body,
        out_type=jax.ShapeDtypeStruct((_B, _D), jnp.float32),
        mesh=plsc.VectorSubcoreMesh(core_axis_name="c", subcore_axis_name="s"),
        scratch_types=[
            pltpu.VMEM((_CPW, _C), jnp.int32),
            pltpu.VMEM((_ROWS, _D), jnp.float32),
            pltpu.VMEM((_ROWS, _D), jnp.float32),
            pltpu.SemaphoreType.DMA,
            pltpu.SemaphoreType.DMA,
            pltpu.SemaphoreType.DMA,
        ],
        compiler_params=pltpu.CompilerParams(use_tc_tiling_on_sc=False),
    )
    return k(codebook, idx2d)


def kernel(indices, codebook):
    idx2d = indices.reshape(_NCHUNK, _C)
    out = _gather(codebook, idx2d)
    return out.reshape(indices.shape + (codebook.shape[-1],))


# speedup vs baseline: 3.8226x; 1.0004x over previous
import functools

import jax
import jax.numpy as jnp
from jax import lax
from jax.experimental import pallas as pl
from jax.experimental.pallas import tpu as pltpu
from jax.experimental.pallas import tpu_sc as plsc

_INFO = plsc.get_sparse_core_info()
_NC = _INFO.num_cores
_NS = _INFO.num_subcores
_NW = _NC * _NS

_B = 128 * 1024
_D = 64
_V = 8192
_C = 128
_NCHUNK = _B // _C
_CPW = _NCHUNK // _NW
_K = 4
_ROWS = _K * _C
_G = _CPW // _K


def _gather_body(codebook_hbm, idx_hbm, out_hbm,
                 idx_v, rows_a, rows_b, gsem, wsem_a, wsem_b):
    wid = lax.axis_index("s") * _NC + lax.axis_index("c")
    row0 = wid * _CPW
    pltpu.sync_copy(idx_hbm.at[pl.ds(row0, _CPW)], idx_v)

    def half(p, g, rows_v, wsem):
        @pl.when(p > 0)
        def _():
            pltpu.make_async_copy(
                rows_v, out_hbm.at[0, pl.ds(0, _ROWS)], wsem).wait()
        descs = [
            pltpu.async_copy(
                codebook_hbm.at[idx_v.at[g * _K + k]],
                rows_v.at[pl.ds(k * _C, _C)],
                gsem,
            )
            for k in range(_K)
        ]
        for d in descs:
            d.wait()
        flat0 = (row0 + g * _K) * _C
        pltpu.async_copy(
            rows_v, out_hbm.at[flat0 // 1024, pl.ds(flat0 % 1024, _ROWS)],
            wsem)

    def step(p, carry):
        half(p, 2 * p, rows_a, wsem_a)
        half(p, 2 * p + 1, rows_b, wsem_b)
        return carry

    lax.fori_loop(0, _G // 2, step, 0)
    pltpu.make_async_copy(
        rows_a, out_hbm.at[0, pl.ds(0, _ROWS)], wsem_a).wait()
    pltpu.make_async_copy(
        rows_b, out_hbm.at[0, pl.ds(0, _ROWS)], wsem_b).wait()


@functools.partial(jax.jit, static_argnames=())
def _gather(codebook, idx2d):
    k = pl.kernel(
        _gather_body,
        out_type=jax.ShapeDtypeStruct((128, 1024, _D), jnp.float32),
        mesh=plsc.VectorSubcoreMesh(core_axis_name="c", subcore_axis_name="s"),
        scratch_types=[
            pltpu.VMEM((_CPW, _C), jnp.int32),
            pltpu.VMEM((_ROWS, _D), jnp.float32),
            pltpu.VMEM((_ROWS, _D), jnp.float32),
            pltpu.SemaphoreType.DMA,
            pltpu.SemaphoreType.DMA,
            pltpu.SemaphoreType.DMA,
        ],
        compiler_params=pltpu.CompilerParams(use_tc_tiling_on_sc=False),
    )
    return k(codebook, idx2d)


def kernel(indices, codebook):
    idx2d = indices.reshape(_NCHUNK, _C)
    return _gather(codebook, idx2d)


# speedup vs baseline: 4.1277x; 1.0798x over previous
import functools

import jax
import jax.numpy as jnp
from jax import lax
from jax.experimental import pallas as pl
from jax.experimental.pallas import tpu as pltpu
from jax.experimental.pallas import tpu_sc as plsc

_INFO = plsc.get_sparse_core_info()
_NC = _INFO.num_cores
_NS = _INFO.num_subcores
_NW = _NC * _NS

_B = 128 * 1024
_D = 64
_DP = 128
_C = 128
_NCHUNK = _B // _C
_CPW = _NCHUNK // _NW
_K = 2
_ROWS = _K * _C
_G = _CPW // _K


def _gather_body(codebook_hbm, idx_hbm, out_hbm,
                 idx_v, rows_ga, rows_gb, rows_c, gsem_a, gsem_b, wsem):
    wid = lax.axis_index("s") * _NC + lax.axis_index("c")
    row0 = wid * _CPW
    pltpu.sync_copy(idx_hbm.at[pl.ds(row0, _CPW)], idx_v)

    def fire_gathers(g, buf, gsem):
        for k in range(_K):
            pltpu.async_copy(
                codebook_hbm.at[idx_v.at[g * _K + k]],
                buf.at[pl.ds(k * _C, _C)],
                gsem,
            )

    def wait_gathers(buf, gsem):
        for k in range(_K):
            pltpu.make_async_copy(
                codebook_hbm.at[idx_v.at[0]],
                buf.at[pl.ds(k * _C, _C)],
                gsem,
            ).wait()

    def wait_write():
        pltpu.make_async_copy(
            rows_c, out_hbm.at[0, pl.ds(0, _ROWS)], wsem).wait()

    def compact(buf):
        def row(r, carry):
            for k in range(_D // 16):
                rows_c[r, pl.ds(k * 16, 16)] = buf[r, pl.ds(k * 16, 16)]
            return carry
        lax.fori_loop(0, _ROWS, row, 0)

    def fire_write(g):
        flat0 = (row0 + g * _K) * _C
        pltpu.async_copy(
            rows_c,
            out_hbm.at[flat0 // 1024, pl.ds(flat0 % 1024, _ROWS)],
            wsem,
        )

    def handle(p, g, buf, gsem, next_g, next_buf, next_gsem, guard_next):
        if guard_next:
            @pl.when(next_g < _G)
            def _():
                fire_gathers(next_g, next_buf, next_gsem)
        else:
            fire_gathers(next_g, next_buf, next_gsem)
        wait_gathers(buf, gsem)

        @pl.when(p > 0)
        def _():
            wait_write()
        compact(buf)
        fire_write(g)

    fire_gathers(0, rows_ga, gsem_a)

    def pair(p, carry):
        handle(p, 2 * p, rows_ga, gsem_a, 2 * p + 1, rows_gb, gsem_b, False)
        handle(p + 1, 2 * p + 1, rows_gb, gsem_b,
               2 * p + 2, rows_ga, gsem_a, True)
        return carry

    lax.fori_loop(0, _G // 2, pair, 0)
    wait_write()


@functools.partial(jax.jit, static_argnames=())
def _gather(codebook_p, idx2d):
    k = pl.kernel(
        _gather_body,
        out_type=jax.ShapeDtypeStruct((128, 1024, _D), jnp.float32),
        mesh=plsc.VectorSubcoreMesh(core_axis_name="c", subcore_axis_name="s"),
        scratch_types=[
            pltpu.VMEM((_CPW, _C), jnp.int32),
            pltpu.VMEM((_ROWS, _DP), jnp.float32),
            pltpu.VMEM((_ROWS, _DP), jnp.float32),
            pltpu.VMEM((_ROWS, _D), jnp.float32),
            pltpu.SemaphoreType.DMA,
            pltpu.SemaphoreType.DMA,
            pltpu.SemaphoreType.DMA,
        ],
        compiler_params=pltpu.CompilerParams(use_tc_tiling_on_sc=True),
    )
    return k(codebook_p, idx2d)


def kernel(indices, codebook):
    idx2d = indices.reshape(_NCHUNK, _C)
    codebook_p = jnp.pad(codebook, ((0, 0), (0, _DP - _D)))
    return _gather(codebook_p, idx2d)
